# Initial kernel scaffold; baseline (speedup 1.0000x reference)
import jax
import jax.numpy as jnp
from jax.experimental import pallas as pl


def kernel(hidden, word_index, word_attention_mask):
    raise NotImplementedError("write your pallas kernel here")


# speedup vs baseline: 1.2688x; 1.2688x over previous
import functools

import jax
import jax.numpy as jnp
from jax import lax
from jax.experimental import pallas as pl
from jax.experimental.pallas import tpu as pltpu
from jax.experimental.pallas import tpu_sc as plsc

B, S, D, W = 4, 4096, 1024, 2048

NC, NS = 2, 16
NW = NC * NS
WK_PER_B = NW // B
WPW = W // WK_PER_B
CHUNK = 64
NCHUNK = WPW // CHUNK


def _build_sc_gather():
    mesh = plsc.VectorSubcoreMesh(core_axis_name="c", subcore_axis_name="s")

    @functools.partial(
        pl.kernel,
        mesh=mesh,
        out_type=jax.ShapeDtypeStruct((B, W + 1, 8, D // 8), jnp.float32),
        scratch_types=[
            pltpu.VMEM((WPW,), jnp.int32),
            pltpu.VMEM((CHUNK, 8, D // 8), jnp.float32),
            pltpu.SemaphoreType.DMA,
        ],
    )
    def sc_gather(hid_hbm, widx_hbm, out_hbm, idx_v, rows_v, sem):
        wid = lax.axis_index("s") * NC + lax.axis_index("c")
        b = wid // WK_PER_B
        wk = wid % WK_PER_B
        pltpu.sync_copy(widx_hbm.at[pl.ds(b * W + wk * WPW, WPW)], idx_v)
        off = b * S + 1
        for i in range(WPW // 16):
            sl = pl.ds(i * 16, 16)
            idx_v[sl] = idx_v[sl] + off
        for c in range(NCHUNK):
            pltpu.async_copy(
                hid_hbm.at[idx_v.at[pl.ds(c * CHUNK, CHUNK)]], rows_v, sem
            ).wait()
            pltpu.sync_copy(
                rows_v, out_hbm.at[b, pl.ds(1 + wk * WPW + c * CHUNK, CHUNK)]
            )

        @pl.when(wk == 0)
        def _():
            pltpu.sync_copy(hid_hbm.at[pl.ds(b * S, 1)], rows_v.at[pl.ds(0, 1)])
            pltpu.sync_copy(rows_v.at[pl.ds(0, 1)], out_hbm.at[b, pl.ds(0, 1)])

    return sc_gather


_sc_gather = _build_sc_gather()


def kernel(hidden, word_index, word_attention_mask):
    hid_flat = hidden.reshape(B * S, 8, D // 8)
    widx_flat = word_index.astype(jnp.int32).reshape(B * W)
    out = _sc_gather(hid_flat, widx_flat)
    return out.reshape(B, W + 1, D), word_attention_mask


# speedup vs baseline: 1.2905x; 1.0171x over previous
import functools

import jax
import jax.numpy as jnp
from jax import lax
from jax.experimental import pallas as pl
from jax.experimental.pallas import tpu as pltpu
from jax.experimental.pallas import tpu_sc as plsc

B, S, D, W = 4, 4096, 1024, 2048

NC, NS = 2, 16
NW = NC * NS
WK_PER_B = NW // B
WPW = W // WK_PER_B
CHUNK = 32
NBUF = 3
NCHUNK = WPW // CHUNK


def _build_sc_gather():
    mesh = plsc.VectorSubcoreMesh(core_axis_name="c", subcore_axis_name="s")

    @functools.partial(
        pl.kernel,
        mesh=mesh,
        out_type=jax.ShapeDtypeStruct((B, W + 1, 8, D // 8), jnp.float32),
        scratch_types=[
            pltpu.VMEM((WPW,), jnp.int32),
            pltpu.VMEM((NBUF, CHUNK, 8, D // 8), jnp.float32),
        ]
        + [pltpu.SemaphoreType.DMA] * (2 * NBUF),
    )
    def sc_gather(hid_hbm, widx_hbm, out_hbm, idx_v, rows_v, *sems):
        gsem, osem = sems[:NBUF], sems[NBUF:]
        wid = lax.axis_index("s") * NC + lax.axis_index("c")
        b = wid // WK_PER_B
        wk = wid % WK_PER_B
        pltpu.sync_copy(widx_hbm.at[pl.ds(b * W + wk * WPW, WPW)], idx_v)
        off = b * S + 1
        for i in range(WPW // 16):
            sl = pl.ds(i * 16, 16)
            idx_v[sl] = idx_v[sl] + off

        def start_gather(c):
            s = c % NBUF
            return pltpu.async_copy(
                hid_hbm.at[idx_v.at[pl.ds(c * CHUNK, CHUNK)]], rows_v.at[s], gsem[s]
            )

        gathers = [start_gather(c) for c in range(NBUF)]
        writes = [None] * NBUF
        for c in range(NCHUNK):
            s = c % NBUF
            gathers[s].wait()
            writes[s] = pltpu.async_copy(
                rows_v.at[s],
                out_hbm.at[b, pl.ds(1 + wk * WPW + c * CHUNK, CHUNK)],
                osem[s],
            )
            if c + NBUF < NCHUNK:
                writes[s].wait()
                gathers[s] = start_gather(c + NBUF)

        for c in range(max(NCHUNK - NBUF, 0), NCHUNK):
            writes[c % NBUF].wait()

        @pl.when(wk == 0)
        def _():
            pltpu.sync_copy(hid_hbm.at[pl.ds(b * S, 1)], rows_v.at[0, pl.ds(0, 1)])
            pltpu.sync_copy(rows_v.at[0, pl.ds(0, 1)], out_hbm.at[b, pl.ds(0, 1)])

    return sc_gather


_sc_gather = _build_sc_gather()


def kernel(hidden, word_index, word_attention_mask):
    hid_flat = hidden.reshape(B * S, 8, D // 8)
    widx_flat = word_index.astype(jnp.int32).reshape(B * W)
    out = _sc_gather(hid_flat, widx_flat)
    return out.reshape(B, W + 1, D), word_attention_mask


# speedup vs baseline: 2.6125x; 2.0245x over previous
import functools

import jax
import jax.numpy as jnp
from jax import lax
from jax.experimental import pallas as pl
from jax.experimental.pallas import tpu as pltpu
from jax.experimental.pallas import tpu_sc as plsc

B, S, D, W = 4, 4096, 1024, 2048

NC, NS = 2, 16
NW = NC * NS
WK_PER_B = NW // B
RPW = (W + 1 + 7) // 8 // WK_PER_B * 8
CHUNK = 32
NBUF = 3
NCHUNK = RPW // CHUNK


def _build_sc_gather():
    mesh = plsc.VectorSubcoreMesh(core_axis_name="c", subcore_axis_name="s")

    @functools.partial(
        pl.kernel,
        mesh=mesh,
        out_type=(
            jax.ShapeDtypeStruct((B, W + 1, D), jnp.float32),
            jax.ShapeDtypeStruct((B, 8, D), jnp.float32),
        ),
        scratch_types=[
            pltpu.VMEM((16 + W + 16,), jnp.int32),
            pltpu.VMEM((RPW + 16,), jnp.int32),
            pltpu.VMEM((NBUF, CHUNK, D), jnp.float32),
            pltpu.VMEM((8, D), jnp.float32),
        ]
        + [pltpu.SemaphoreType.DMA] * (2 * NBUF),
    )
    def sc_gather(
        hid_hbm, widx_hbm, out_hbm, tail_hbm, widx_v, idx_v, rows_v, rows8_v, *sems
    ):
        gsem, osem = sems[:NBUF], sems[NBUF:]
        wid = lax.axis_index("s") * NC + lax.axis_index("c")
        b = wid // WK_PER_B
        wk = wid % WK_PER_B
        pltpu.sync_copy(widx_hbm.at[pl.ds(b * W, W)], widx_v.at[pl.ds(16, W)])
        lanes = lax.iota(jnp.int32, 16)
        widx_v[pl.ds(0, 16)] = jnp.where(lanes == 15, -1, 0)
        widx_v[pl.ds(16 + W, 16)] = jnp.zeros((16,), jnp.int32)

        base_j = wk * RPW
        hid0 = b * S
        for i in range(RPW // 16 + 1):
            v = widx_v[pl.ds(15 + base_j + i * 16, 16)]
            idx_v[pl.ds(i * 16, 16)] = v + (hid0 + 1)

        def start_gather(c, rows=CHUNK):
            s = c % NBUF
            return pltpu.async_copy(
                hid_hbm.at[idx_v.at[pl.ds(c * CHUNK, rows)]],
                rows_v.at[s, pl.ds(0, rows)],
                gsem[s],
            )

        gathers = [start_gather(c) for c in range(NBUF)]
        writes = [None] * NBUF
        for c in range(NCHUNK):
            s = c % NBUF
            gathers[s].wait()
            writes[s] = pltpu.async_copy(
                rows_v.at[s],
                out_hbm.at[b, pl.ds(base_j + c * CHUNK, CHUNK)],
                osem[s],
            )
            if c + NBUF < NCHUNK:
                writes[s].wait()
                gathers[s] = start_gather(c + NBUF)
        for c in range(max(NCHUNK - NBUF, 0), NCHUNK):
            writes[c % NBUF].wait()

        @pl.when(wk == WK_PER_B - 1)
        def _():
            sl = pltpu.async_copy(
                hid_hbm.at[idx_v.at[pl.ds(RPW, 8)]], rows8_v, gsem[0]
            )
            sl.wait()
            pltpu.sync_copy(rows8_v, tail_hbm.at[b])

    return sc_gather


_sc_gather = _build_sc_gather()


def kernel(hidden, word_index, word_attention_mask):
    hid_flat = hidden.reshape(B * S, D)
    widx_flat = word_index.astype(jnp.int32).reshape(B * W)
    out, tail = _sc_gather(hid_flat, widx_flat)
    out = jax.lax.dynamic_update_slice(out, tail[:, :1, :], (0, W, 0))
    return out, word_attention_mask
